# baseline (speedup 1.0000x reference)
import functools

import jax
import jax.numpy as jnp
from jax import lax
from jax.experimental import pallas as pl
from jax.experimental.pallas import tpu as pltpu
from jax.experimental.pallas import tpu_sc as plsc

NUM_CORES = 2
NUM_SUBCORES = 16
NUM_WORKERS = NUM_CORES * NUM_SUBCORES
LANES = 16
CS = 8
SETS = 3
UNROLL = 4


def kernel(x, embedding):
    B, S, D = x.shape
    s_per_w = S // NUM_WORKERS
    chunks = s_per_w // CS
    cd = CS * D
    x1 = x.reshape(-1)
    e1 = embedding.reshape(-1)

    mesh = plsc.VectorSubcoreMesh(core_axis_name="c", subcore_axis_name="s")

    scratch = []
    for _ in range(SETS):
        scratch.append(pltpu.VMEM((cd,), jnp.float32))
        for _ in range(B):
            scratch.append(pltpu.VMEM((cd,), jnp.float32))
        scratch.append(pltpu.SemaphoreType.DMA)
        scratch.append(pltpu.SemaphoreType.DMA)

    @functools.partial(
        pl.kernel,
        mesh=mesh,
        out_type=jax.ShapeDtypeStruct((B * S * D,), x.dtype),
        scratch_types=scratch,
    )
    def body(x_hbm, emb_hbm, out_hbm, *scr):
        per = B + 3
        sets = [scr[i * per:(i + 1) * per] for i in range(SETS)]
        wid = lax.axis_index("s") * NUM_CORES + lax.axis_index("c")
        s0 = wid * s_per_w

        def issue_loads(g):
            eb = sets[g % SETS][0]
            xbs = sets[g % SETS][1:1 + B]
            lsem = sets[g % SETS][B + 1]
            hs = [pltpu.async_copy(
                emb_hbm.at[pl.ds((s0 + g * CS) * D, cd)], eb, lsem)]
            for b in range(B):
                hs.append(pltpu.async_copy(
                    x_hbm.at[pl.ds((b * S + s0 + g * CS) * D, cd)],
                    xbs[b], lsem))
            return hs

        def issue_stores(g):
            xbs = sets[g % SETS][1:1 + B]
            ssem = sets[g % SETS][B + 2]
            hs = []
            for b in range(B):
                hs.append(pltpu.async_copy(
                    xbs[b],
                    out_hbm.at[pl.ds((b * S + s0 + g * CS) * D, cd)], ssem))
            return hs

        def compute(g):
            eb = sets[g % SETS][0]
            xbs = sets[g % SETS][1:1 + B]

            def iter_body(j, carry):
                base = j * (LANES * UNROLL)
                for k in range(UNROLL):
                    sl = pl.ds(base + k * LANES, LANES)
                    e = eb[sl]
                    for b in range(B):
                        xbs[b][sl] = xbs[b][sl] + e
                return carry

            lax.fori_loop(0, cd // (LANES * UNROLL), iter_body, 0)

        load_hs = {}
        store_hs = {}
        for g in range(min(SETS - 1, chunks)):
            load_hs[g] = issue_loads(g)
        for g in range(chunks):
            if g >= 2:
                for h in store_hs.pop(g - 2):
                    h.wait()
            if g + SETS - 1 < chunks:
                load_hs[g + SETS - 1] = issue_loads(g + SETS - 1)
            for h in load_hs.pop(g):
                h.wait()
            compute(g)
            store_hs[g] = issue_stores(g)
        for g in sorted(store_hs):
            for h in store_hs[g]:
                h.wait()

    out = body(x1, e1)
    return out.reshape(B, S, D)
